# Initial kernel scaffold; baseline (speedup 1.0000x reference)
import jax
import jax.numpy as jnp
from jax.experimental import pallas as pl


def kernel(word, pos1, pos2, word_table, pos1_table, pos2_table):
    raise NotImplementedError("write your pallas kernel here")


# speedup vs baseline: 3.8561x; 3.8561x over previous
import functools

import jax
import jax.numpy as jnp
from jax import lax
from jax.experimental import pallas as pl
from jax.experimental.pallas import tpu as pltpu
from jax.experimental.pallas import tpu_sc as plsc

_NC = 2
_NS = 16
_NW = _NC * _NS

_B = 4096
_S = 200
_N = _B * _S
_WD = 50
_PD = 5
_OD = _WD + 2 * _PD

_WDP = 56
_PDP = 8

_IB = 128
_C = 512
_JPC = _C // _IB
_NPW = _N // _NW
_NCHUNK = _NPW // _C


def _gather_body(word_hbm, pos1_hbm, pos2_hbm, wtab_hbm, p1tab_hbm,
                 p2tab_hbm, wout_hbm, p1out_hbm, p2out_hbm,
                 widx_v, p1idx_v, p2idx_v, wrow_v, p1row_v, p2row_v,
                 sem, osem):
    wid = lax.axis_index("s") * _NC + lax.axis_index("c")
    row0 = wid * _NPW

    def chunk_body(ci, _):
        rbase = row0 + ci * _C
        ibase = rbase // _IB
        pltpu.sync_copy(word_hbm.at[pl.ds(ibase, _JPC)], widx_v)
        pltpu.sync_copy(pos1_hbm.at[pl.ds(ibase, _JPC)], p1idx_v)
        pltpu.sync_copy(pos2_hbm.at[pl.ds(ibase, _JPC)], p2idx_v)
        for j in range(_JPC):
            rows = pl.ds(j * _IB, _IB)
            pltpu.async_copy(
                wtab_hbm.at[widx_v.at[j]], wrow_v.at[rows], sem)
            pltpu.async_copy(
                p1tab_hbm.at[p1idx_v.at[j]], p1row_v.at[rows], sem)
            pltpu.async_copy(
                p2tab_hbm.at[p2idx_v.at[j]], p2row_v.at[rows], sem)
        for j in range(_JPC):
            rows = pl.ds(j * _IB, _IB)
            pltpu.make_async_copy(
                wtab_hbm.at[widx_v.at[j]], wrow_v.at[rows], sem).wait()
            pltpu.make_async_copy(
                p1tab_hbm.at[p1idx_v.at[j]], p1row_v.at[rows], sem).wait()
            pltpu.make_async_copy(
                p2tab_hbm.at[p2idx_v.at[j]], p2row_v.at[rows], sem).wait()
        pltpu.async_copy(wrow_v, wout_hbm.at[pl.ds(rbase, _C)], osem)
        pltpu.async_copy(p1row_v, p1out_hbm.at[pl.ds(rbase, _C)], osem)
        pltpu.async_copy(p2row_v, p2out_hbm.at[pl.ds(rbase, _C)], osem)
        pltpu.make_async_copy(
            wrow_v, wout_hbm.at[pl.ds(rbase, _C)], osem).wait()
        pltpu.make_async_copy(
            p1row_v, p1out_hbm.at[pl.ds(rbase, _C)], osem).wait()
        pltpu.make_async_copy(
            p2row_v, p2out_hbm.at[pl.ds(rbase, _C)], osem).wait()
        return ()

    lax.fori_loop(0, _NCHUNK, chunk_body, ())


_RB = 4096


def _concat_body(w_ref, p1_ref, p2_ref, out_ref):
    out_ref[...] = jnp.concatenate(
        [w_ref[:, :_WD], p1_ref[:, :_PD], p2_ref[:, :_PD]], axis=1)


@jax.jit
def _embed(word, pos1, pos2, word_table, pos1_table, pos2_table):
    mesh = plsc.VectorSubcoreMesh(
        core_axis_name="c", subcore_axis_name="s",
        num_cores=_NC, num_subcores=_NS)
    gather = pl.kernel(
        _gather_body,
        out_type=(
            jax.ShapeDtypeStruct((_N, _WDP), jnp.float32),
            jax.ShapeDtypeStruct((_N, _PDP), jnp.float32),
            jax.ShapeDtypeStruct((_N, _PDP), jnp.float32),
        ),
        mesh=mesh,
        scratch_types=[
            pltpu.VMEM((_JPC, _IB), jnp.int32),
            pltpu.VMEM((_JPC, _IB), jnp.int32),
            pltpu.VMEM((_JPC, _IB), jnp.int32),
            pltpu.VMEM((_C, _WDP), jnp.float32),
            pltpu.VMEM((_C, _PDP), jnp.float32),
            pltpu.VMEM((_C, _PDP), jnp.float32),
            pltpu.SemaphoreType.DMA,
            pltpu.SemaphoreType.DMA,
        ],
        compiler_params=pltpu.CompilerParams(use_tc_tiling_on_sc=False),
    )
    wtab = jnp.pad(word_table, ((0, 0), (0, _WDP - _WD)))
    p1tab = jnp.pad(pos1_table, ((0, 0), (0, _PDP - _PD)))
    p2tab = jnp.pad(pos2_table, ((0, 0), (0, _PDP - _PD)))
    wrows, p1rows, p2rows = gather(
        word.reshape(_N // _IB, _IB), pos1.reshape(_N // _IB, _IB),
        pos2.reshape(_N // _IB, _IB), wtab, p1tab, p2tab)

    out = pl.pallas_call(
        _concat_body,
        grid=(_N // _RB,),
        in_specs=[
            pl.BlockSpec((_RB, _WDP), lambda i: (i, 0)),
            pl.BlockSpec((_RB, _PDP), lambda i: (i, 0)),
            pl.BlockSpec((_RB, _PDP), lambda i: (i, 0)),
        ],
        out_specs=pl.BlockSpec((_RB, _OD), lambda i: (i, 0)),
        out_shape=jax.ShapeDtypeStruct((_N, _OD), jnp.float32),
    )(wrows, p1rows, p2rows)
    return out.reshape(_B, _S, _OD)


def kernel(word, pos1, pos2, word_table, pos1_table, pos2_table):
    return _embed(word, pos1, pos2, word_table, pos1_table, pos2_table)


# speedup vs baseline: 5.5744x; 1.4456x over previous
import functools

import jax
import jax.numpy as jnp
from jax import lax
from jax.experimental import pallas as pl
from jax.experimental.pallas import tpu as pltpu
from jax.experimental.pallas import tpu_sc as plsc

_NC = 2
_NS = 16
_NW = _NC * _NS

_B = 4096
_S = 200
_N = _B * _S
_WD = 50
_PD = 5
_OD = _WD + 2 * _PD
_WDP = 56
_PDP = 8

_IB = 128
_C = 256
_JPC = _C // _IB
_NPW = _N // _NW
_NCHUNK = _NPW // _C
_U = 4


def _ivec(vals):
    lanes = lax.broadcasted_iota(jnp.int32, (16,), 0)
    out = lanes * 0
    for i, x in enumerate(vals):
        out = jnp.where(lanes == i, jnp.int32(x), out)
    return out


def _make_body():
    def body(word_hbm, pos1_hbm, pos2_hbm, wtab_hbm, p1tab_hbm, p2tab_hbm,
             out_hbm,
             widx0, p1idx0, p2idx0, wrow0, p1row0, p2row0, outv0,
             widx1, p1idx1, p2idx1, wrow1, p1row1, p2row1, outv1,
             gsem0, gsem1, osem0, osem1):
        wid = lax.axis_index("s") * _NC + lax.axis_index("c")
        row0 = wid * _NPW
        lanes = lax.broadcasted_iota(jnp.int32, (16,), 0)
        w_rowoff = _ivec([0, 0, 0, 0, 0, 0, 0, 0, 0, 0, 0, 0, 1, 1, 1, 1])
        w_col = _ivec([48, 49, 0, 0, 0, 0, 0, 0, 0, 0, 0, 0, 0, 1, 2, 3])
        p_col = _ivec([0, 0, 0, 1, 2, 3, 4, 0, 1, 2, 3, 4, 0, 0, 0, 0])
        m_w = jnp.logical_or(lanes < 2, lanes >= 12)
        m_p1 = jnp.logical_and(lanes >= 2, lanes < 7)
        zeros = lanes * 0

        bufs = ((widx0, p1idx0, p2idx0, wrow0, p1row0, p2row0, outv0,
                 gsem0, osem0),
                (widx1, p1idx1, p2idx1, wrow1, p1row1, p2row1, outv1,
                 gsem1, osem1))

        def load_and_fire(ci, b):
            widx, p1idx, p2idx, wrow, p1row, p2row, _, gsem, _ = bufs[b]
            ibase = (row0 + ci * _C) // _IB
            pltpu.sync_copy(word_hbm.at[pl.ds(ibase, _JPC)], widx)
            pltpu.sync_copy(pos1_hbm.at[pl.ds(ibase, _JPC)], p1idx)
            pltpu.sync_copy(pos2_hbm.at[pl.ds(ibase, _JPC)], p2idx)
            for j in range(_JPC):
                rows = pl.ds(j * _IB, _IB)
                pltpu.async_copy(
                    wtab_hbm.at[widx.at[j]], wrow.at[rows], gsem)
                pltpu.async_copy(
                    p1tab_hbm.at[p1idx.at[j]], p1row.at[rows], gsem)
                pltpu.async_copy(
                    p2tab_hbm.at[p2idx.at[j]], p2row.at[rows], gsem)

        def wait_gathers(b):
            widx, p1idx, p2idx, wrow, p1row, p2row, _, gsem, _ = bufs[b]
            for j in range(_JPC):
                rows = pl.ds(j * _IB, _IB)
                pltpu.make_async_copy(
                    wtab_hbm.at[widx.at[j]], wrow.at[rows], gsem).wait()
                pltpu.make_async_copy(
                    p1tab_hbm.at[p1idx.at[j]], p1row.at[rows], gsem).wait()
                pltpu.make_async_copy(
                    p2tab_hbm.at[p2idx.at[j]], p2row.at[rows], gsem).wait()

        def assemble(b):
            _, _, _, wrow, p1row, p2row, outv, _, _ = bufs[b]

            def rows_body(g, _):
                r_base = g * _U
                for u in range(_U):
                    r = r_base + u
                    ob = r * _OD
                    outv[pl.ds(ob, 16)] = wrow[r, pl.ds(0, 16)]
                    outv[pl.ds(ob + 16, 16)] = wrow[r, pl.ds(16, 16)]
                    outv[pl.ds(ob + 32, 16)] = wrow[r, pl.ds(32, 16)]
                    rv = zeros + r
                    gw = plsc.load_gather(wrow, [rv + w_rowoff, w_col])
                    gp1 = plsc.load_gather(p1row, [rv, p_col])
                    gp2 = plsc.load_gather(p2row, [rv, p_col])
                    tail = jnp.where(m_w, gw, jnp.where(m_p1, gp1, gp2))
                    outv[pl.ds(ob + 48, 16)] = tail
                return ()

            lax.fori_loop(0, _C // _U, rows_body, ())

        def write_out(ci, b):
            outv, osem = bufs[b][6], bufs[b][8]
            obase = (row0 + ci * _C) * _OD
            pltpu.async_copy(outv.at[pl.ds(0, _C * _OD)],
                             out_hbm.at[pl.ds(obase, _C * _OD)], osem)

        def wait_out(ci, b):
            outv, osem = bufs[b][6], bufs[b][8]
            obase = (row0 + ci * _C) * _OD
            pltpu.make_async_copy(
                outv.at[pl.ds(0, _C * _OD)],
                out_hbm.at[pl.ds(obase, _C * _OD)], osem).wait()

        load_and_fire(0, 0)

        def pair_body(pi, _):
            c0 = pi * 2
            wait_gathers(0)
            load_and_fire(c0 + 1, 1)
            lax.cond(pi > 0, lambda: wait_out(c0 - 1, 0), lambda: None)
            assemble(0)
            write_out(c0, 0)
            wait_gathers(1)
            lax.cond(pi + 1 < _NCHUNK // 2,
                     lambda: load_and_fire(c0 + 2, 0), lambda: None)
            lax.cond(pi > 0, lambda: wait_out(c0, 1), lambda: None)
            assemble(1)
            write_out(c0 + 1, 1)
            return ()

        lax.fori_loop(0, _NCHUNK // 2, pair_body, ())
        wait_out(_NCHUNK - 2, 0)
        wait_out(_NCHUNK - 1, 1)

    return body


@jax.jit
def _embed(word, pos1, pos2, word_table, pos1_table, pos2_table):
    mesh = plsc.VectorSubcoreMesh(
        core_axis_name="c", subcore_axis_name="s",
        num_cores=_NC, num_subcores=_NS)
    run = pl.kernel(
        _make_body(),
        out_type=jax.ShapeDtypeStruct((_N * _OD,), jnp.float32),
        mesh=mesh,
        scratch_types=[
            pltpu.VMEM((_JPC, _IB), jnp.int32),
            pltpu.VMEM((_JPC, _IB), jnp.int32),
            pltpu.VMEM((_JPC, _IB), jnp.int32),
            pltpu.VMEM((_C + 8, _WDP), jnp.float32),
            pltpu.VMEM((_C, _PDP), jnp.float32),
            pltpu.VMEM((_C, _PDP), jnp.float32),
            pltpu.VMEM((_C * _OD + 16,), jnp.float32),
            pltpu.VMEM((_JPC, _IB), jnp.int32),
            pltpu.VMEM((_JPC, _IB), jnp.int32),
            pltpu.VMEM((_JPC, _IB), jnp.int32),
            pltpu.VMEM((_C + 8, _WDP), jnp.float32),
            pltpu.VMEM((_C, _PDP), jnp.float32),
            pltpu.VMEM((_C, _PDP), jnp.float32),
            pltpu.VMEM((_C * _OD + 16,), jnp.float32),
            pltpu.SemaphoreType.DMA,
            pltpu.SemaphoreType.DMA,
            pltpu.SemaphoreType.DMA,
            pltpu.SemaphoreType.DMA,
        ],
        compiler_params=pltpu.CompilerParams(
            use_tc_tiling_on_sc=False, needs_layout_passes=False),
    )
    wtab = jnp.pad(word_table, ((0, 0), (0, _WDP - _WD)))
    p1tab = jnp.pad(pos1_table, ((0, 0), (0, _PDP - _PD)))
    p2tab = jnp.pad(pos2_table, ((0, 0), (0, _PDP - _PD)))
    out = run(word.reshape(_N // _IB, _IB), pos1.reshape(_N // _IB, _IB),
              pos2.reshape(_N // _IB, _IB), wtab, p1tab, p2tab)
    return out.reshape(_B, _S, _OD)


def kernel(word, pos1, pos2, word_table, pos1_table, pos2_table):
    return _embed(word, pos1, pos2, word_table, pos1_table, pos2_table)


# speedup vs baseline: 7.3239x; 1.3139x over previous
import functools

import jax
import jax.numpy as jnp
from jax import lax
from jax.experimental import pallas as pl
from jax.experimental.pallas import tpu as pltpu
from jax.experimental.pallas import tpu_sc as plsc

_NC = 2
_NS = 16
_NW = _NC * _NS

_B = 4096
_S = 200
_N = _B * _S
_WD = 50
_PD = 5
_OD = _WD + 2 * _PD
_WDP = 56
_PDP = 8

_IB = 128
_C = 256
_JPC = _C // _IB
_NPW = _N // _NW
_NCHUNK = _NPW // _C
_SUP = 10
_NSUP = _NCHUNK // _SUP
_U = 4


def _ivec(vals):
    lanes = lax.broadcasted_iota(jnp.int32, (16,), 0)
    out = lanes * 0
    for i, x in enumerate(vals):
        out = jnp.where(lanes == i, jnp.int32(x), out)
    return out


def _make_body():
    def body(word_hbm, pos1_hbm, pos2_hbm, wtab_hbm, p1tab_hbm, p2tab_hbm,
             out_hbm,
             sidxw, sidxp1, sidxp2,
             wrow0, pbuf0, outv0, wrow1, pbuf1, outv1,
             gsem0, gsem1, osem0, osem1):
        wid = lax.axis_index("s") * _NC + lax.axis_index("c")
        row0 = wid * _NPW
        lanes = lax.broadcasted_iota(jnp.int32, (16,), 0)
        w_col = _ivec([44, 45, 46, 47, 48, 49] + [0] * 10)
        pos_rowoff = _ivec([0] * 11 + [_C] * 5)
        pos_col = _ivec([0] * 6 + [0, 1, 2, 3, 4, 0, 1, 2, 3, 4])
        m_w = lanes < 6
        zeros = lanes * 0

        bufs = ((wrow0, pbuf0, outv0, gsem0, osem0),
                (wrow1, pbuf1, outv1, gsem1, osem1))

        def load_idx(si):
            base = row0 + si * _SUP * _C
            pltpu.sync_copy(word_hbm.at[pl.ds(base, _SUP * _C)], sidxw)
            pltpu.sync_copy(pos1_hbm.at[pl.ds(base, _SUP * _C)], sidxp1)
            pltpu.sync_copy(pos2_hbm.at[pl.ds(base, _SUP * _C)], sidxp2)

        def fire(cis, b):
            wrow, pbuf, _, gsem, _ = bufs[b]
            for j in range(_JPC):
                off = cis * _C + j * _IB
                rows = pl.ds(j * _IB, _IB)
                pltpu.async_copy(
                    wtab_hbm.at[sidxw.at[pl.ds(off, _IB)]],
                    wrow.at[rows], gsem)
                pltpu.async_copy(
                    p1tab_hbm.at[sidxp1.at[pl.ds(off, _IB)]],
                    pbuf.at[rows], gsem)
                pltpu.async_copy(
                    p2tab_hbm.at[sidxp2.at[pl.ds(off, _IB)]],
                    pbuf.at[pl.ds(_C + j * _IB, _IB)], gsem)

        def wait_gathers(cis, b):
            wrow, pbuf, _, gsem, _ = bufs[b]
            for j in range(_JPC):
                off = cis * _C + j * _IB
                rows = pl.ds(j * _IB, _IB)
                pltpu.make_async_copy(
                    wtab_hbm.at[sidxw.at[pl.ds(off, _IB)]],
                    wrow.at[rows], gsem).wait()
                pltpu.make_async_copy(
                    p1tab_hbm.at[sidxp1.at[pl.ds(off, _IB)]],
                    pbuf.at[rows], gsem).wait()
                pltpu.make_async_copy(
                    p2tab_hbm.at[sidxp2.at[pl.ds(off, _IB)]],
                    pbuf.at[pl.ds(_C + j * _IB, _IB)], gsem).wait()

        def assemble(b):
            wrow, pbuf, outv, _, _ = bufs[b]

            def rows_body(g, _):
                r_base = g * _U
                for u in range(_U):
                    r = r_base + u
                    outv[r, pl.ds(0, 16)] = wrow[r, pl.ds(0, 16)]
                    outv[r, pl.ds(16, 16)] = wrow[r, pl.ds(16, 16)]
                    outv[r, pl.ds(28, 16)] = wrow[r, pl.ds(28, 16)]
                    rv = zeros + r
                    gw = plsc.load_gather(wrow, [rv, w_col])
                    gp = plsc.load_gather(pbuf, [rv + pos_rowoff, pos_col])
                    outv[r, pl.ds(44, 16)] = jnp.where(m_w, gw, gp)
                return ()

            lax.fori_loop(0, _C // _U, rows_body, ())

        def write_out(cg, b):
            _, _, outv, _, osem = bufs[b]
            rbase = row0 + cg * _C
            pltpu.async_copy(outv, out_hbm.at[pl.ds(rbase, _C)], osem)

        def wait_out(cg, b):
            _, _, outv, _, osem = bufs[b]
            rbase = row0 + cg * _C
            pltpu.make_async_copy(
                outv, out_hbm.at[pl.ds(rbase, _C)], osem).wait()

        def sup_body(si, _):
            load_idx(si)
            fire(0, 0)
            for pj in range(_SUP // 2):
                cg0 = si * _SUP + pj * 2
                wait_gathers(pj * 2, 0)
                fire(pj * 2 + 1, 1)
                lax.cond(cg0 >= 2, lambda: wait_out(cg0 - 2, 0),
                         lambda: None)
                assemble(0)
                write_out(cg0, 0)
                wait_gathers(pj * 2 + 1, 1)
                if pj < _SUP // 2 - 1:
                    fire(pj * 2 + 2, 0)
                lax.cond(cg0 >= 1, lambda: wait_out(cg0 - 1, 1),
                         lambda: None)
                assemble(1)
                write_out(cg0 + 1, 1)
            return ()

        lax.fori_loop(0, _NSUP, sup_body, ())
        wait_out(_NCHUNK - 2, 0)
        wait_out(_NCHUNK - 1, 1)

    return body


@jax.jit
def _embed(word, pos1, pos2, word_table, pos1_table, pos2_table):
    mesh = plsc.VectorSubcoreMesh(
        core_axis_name="c", subcore_axis_name="s",
        num_cores=_NC, num_subcores=_NS)
    run = pl.kernel(
        _make_body(),
        out_type=jax.ShapeDtypeStruct((_N, _OD), jnp.float32),
        mesh=mesh,
        scratch_types=[
            pltpu.VMEM((_SUP * _C,), jnp.int32),
            pltpu.VMEM((_SUP * _C,), jnp.int32),
            pltpu.VMEM((_SUP * _C,), jnp.int32),
            pltpu.VMEM((_C, _WDP), jnp.float32),
            pltpu.VMEM((2 * _C, _PDP), jnp.float32),
            pltpu.VMEM((_C, _OD), jnp.float32),
            pltpu.VMEM((_C, _WDP), jnp.float32),
            pltpu.VMEM((2 * _C, _PDP), jnp.float32),
            pltpu.VMEM((_C, _OD), jnp.float32),
            pltpu.SemaphoreType.DMA,
            pltpu.SemaphoreType.DMA,
            pltpu.SemaphoreType.DMA,
            pltpu.SemaphoreType.DMA,
        ],
        compiler_params=pltpu.CompilerParams(
            use_tc_tiling_on_sc=False, needs_layout_passes=False),
    )
    wtab = jnp.pad(word_table, ((0, 0), (0, _WDP - _WD)))
    p1tab = jnp.pad(pos1_table, ((0, 0), (0, _PDP - _PD)))
    p2tab = jnp.pad(pos2_table, ((0, 0), (0, _PDP - _PD)))
    out = run(word.reshape(_N), pos1.reshape(_N), pos2.reshape(_N),
              wtab, p1tab, p2tab)
    return out.reshape(_B, _S, _OD)


def kernel(word, pos1, pos2, word_table, pos1_table, pos2_table):
    return _embed(word, pos1, pos2, word_table, pos1_table, pos2_table)
